# baseline (speedup 1.0000x reference)
import jax
import jax.numpy as jnp
from jax import lax
from jax.experimental import pallas as pl
from jax.experimental.pallas import tpu as pltpu
from jax.experimental.pallas import tpu_sc as plsc

_BN = 8192
_CP = 16
_NC = 10
_NW = 32
_L = 16


def _tc_body(emb_ref, wwi_ref, wqi_ref, bwi_ref, bqi_ref, bi_ref, qtot_ref,
             wi_ref, qt_ref, f_ref, acc_ref):
    b = pl.program_id(0)
    nb = pl.num_programs(0)
    nsys = qtot_ref.shape[0]
    bn = emb_ref.shape[0]
    dn = (((0,), (1,)), ((), ()))
    yw = lax.dot_general(wwi_ref[...], emb_ref[...], dn,
                         preferred_element_type=jnp.float32)
    yq = lax.dot_general(wqi_ref[...], emb_ref[...], dn,
                         preferred_element_type=jnp.float32)
    ones_row = jnp.ones((1, bn), jnp.float32)
    do = (((0,), (0,)), ((), ()))
    yw = yw + lax.dot_general(bwi_ref[...][None, :], ones_row, do,
                              preferred_element_type=jnp.float32)
    yq = yq + lax.dot_general(bqi_ref[...][None, :], ones_row, do,
                              preferred_element_type=jnp.float32)
    wi = jnp.maximum(yw, 0.0) + jnp.log1p(jnp.exp(-jnp.abs(yw)))
    wi_ref[...] = wi
    qt_ref[...] = yq

    sysid = lax.broadcasted_iota(jnp.int32, (nsys, bn), 0)
    masks_t = (bi_ref[...][None, :] == sysid).astype(jnp.float32)
    dm = (((1,), (1,)), ((), ()))
    part_w = lax.dot_general(wi, masks_t, dm,
                             preferred_element_type=jnp.float32)
    part_q = lax.dot_general(yq, masks_t, dm,
                             preferred_element_type=jnp.float32)
    part = jnp.concatenate([part_w, part_q], axis=0)

    @pl.when(b == 0)
    def _init():
        acc_ref[...] = part

    @pl.when(b != 0)
    def _accum():
        acc_ref[...] += part

    @pl.when(b == nb - 1)
    def _finish():
        acc = acc_ref[...]
        wtot = acc[:_NC, :]
        qtot = acc[_NC:, :]
        f_ref[...] = (qtot_ref[...][None, :] - qtot) / wtot


def _sc_body(wi_hbm, qt_hbm, bi_hbm, f_hbm, out_hbm,
             bi_v, wi_v, qt_v, f_v, s0, s1, s2, s3):
    apw = wi_v.shape[1]
    wid = lax.axis_index("s") * 2 + lax.axis_index("c")
    base = wid * apw
    c0 = pltpu.async_copy(f_hbm, f_v, s0)
    c1 = pltpu.async_copy(bi_hbm.at[pl.ds(base, apw)], bi_v, s1)
    c2 = pltpu.async_copy(wi_hbm.at[:, pl.ds(base, apw)], wi_v, s2)
    c3 = pltpu.async_copy(qt_hbm.at[:, pl.ds(base, apw)], qt_v, s3)
    c0.wait()
    c1.wait()
    c2.wait()
    c3.wait()

    cvecs = [jnp.full((_L,), c, jnp.int32) for c in range(_NC)]

    @plsc.parallel_loop(0, apw, step=_L, unroll=4)
    def _combine(i):
        sl = pl.ds(i, _L)
        bi_vec = bi_v[sl]
        for c in range(_NC):
            fvals = plsc.load_gather(f_v, [cvecs[c], bi_vec])
            qt_v[c, sl] = qt_v[c, sl] + wi_v[c, sl] * fvals

    pltpu.sync_copy(qt_v, out_hbm.at[:, pl.ds(base, apw)])


def kernel(embedding, coordinates, batch_index, natoms, total_charge,
           W_wi, b_wi, W_qi, b_qi):
    n, d = embedding.shape
    nsys = natoms.shape[0]
    nc = W_wi.shape[1]
    apw = n // _NW

    nb = n // _BN
    wi_t, qt_t, f_t = pl.pallas_call(
        _tc_body,
        grid=(nb,),
        in_specs=[
            pl.BlockSpec((_BN, d), lambda b: (b, 0)),
            pl.BlockSpec((d, nc), lambda b: (0, 0)),
            pl.BlockSpec((d, nc), lambda b: (0, 0)),
            pl.BlockSpec((nc,), lambda b: (0,)),
            pl.BlockSpec((nc,), lambda b: (0,)),
            pl.BlockSpec((_BN,), lambda b: (b,)),
            pl.BlockSpec((nsys,), lambda b: (0,)),
        ],
        out_specs=[
            pl.BlockSpec((_NC, _BN), lambda b: (0, b)),
            pl.BlockSpec((_NC, _BN), lambda b: (0, b)),
            pl.BlockSpec((_NC, nsys), lambda b: (0, 0)),
        ],
        out_shape=[
            jax.ShapeDtypeStruct((_NC, n), jnp.float32),
            jax.ShapeDtypeStruct((_NC, n), jnp.float32),
            jax.ShapeDtypeStruct((_NC, nsys), jnp.float32),
        ],
        scratch_shapes=[pltpu.VMEM((2 * _NC, nsys), jnp.float32)],
        compiler_params=pltpu.CompilerParams(
            fuse_transposed_lhs_in_matmul=True),
    )(embedding, W_wi, W_qi, b_wi, b_qi, batch_index, total_charge)

    sc_fn = pl.kernel(
        _sc_body,
        out_type=jax.ShapeDtypeStruct((_NC, n), jnp.float32),
        mesh=plsc.VectorSubcoreMesh(core_axis_name="c", subcore_axis_name="s"),
        compiler_params=pltpu.CompilerParams(needs_layout_passes=False),
        scratch_types=[
            pltpu.VMEM((apw,), jnp.int32),
            pltpu.VMEM((_NC, apw), jnp.float32),
            pltpu.VMEM((_NC, apw), jnp.float32),
            pltpu.VMEM((_NC, nsys), jnp.float32),
            pltpu.SemaphoreType.DMA,
            pltpu.SemaphoreType.DMA,
            pltpu.SemaphoreType.DMA,
            pltpu.SemaphoreType.DMA,
        ],
    )
    q_t = sc_fn(wi_t, qt_t, batch_index, f_t)
    return q_t.T


# speedup vs baseline: 1.0145x; 1.0145x over previous
import jax
import jax.numpy as jnp
from jax import lax
from jax.experimental import pallas as pl
from jax.experimental.pallas import tpu as pltpu
from jax.experimental.pallas import tpu_sc as plsc

_BN = 16384
_CP = 16
_NC = 10
_NW = 32
_L = 16


def _tc_body(emb_ref, wt_ref, b_ref, bi_ref, qtot_ref,
             wi_ref, qt_ref, f_ref, acc_ref):
    b = pl.program_id(0)
    nb = pl.num_programs(0)
    nsys = qtot_ref.shape[0]
    bn = emb_ref.shape[0]
    y = lax.dot_general(wt_ref[...], emb_ref[...], (((1,), (1,)), ((), ())),
                        preferred_element_type=jnp.float32)
    y = y + b_ref[...]
    yw = y[:_CP, :]
    wi = jnp.maximum(yw, 0.0) + jnp.log1p(jnp.exp(-jnp.abs(yw)))
    qt = y[_CP:, :]
    wi_ref[...] = wi[:_NC, :]
    qt_ref[...] = qt[:_NC, :]

    sysid = lax.broadcasted_iota(jnp.int32, (nsys, bn), 0)
    masks_t = (bi_ref[...][None, :] == sysid).astype(jnp.float32)
    ywq = jnp.concatenate([wi, qt], axis=0)
    part = lax.dot_general(ywq, masks_t, (((1,), (1,)), ((), ())),
                           preferred_element_type=jnp.float32)

    @pl.when(b == 0)
    def _init():
        acc_ref[...] = part

    @pl.when(b != 0)
    def _accum():
        acc_ref[...] += part

    @pl.when(b == nb - 1)
    def _finish():
        acc = acc_ref[...]
        wtot = acc[:_CP, :]
        qtot = acc[_CP:, :]
        f = (qtot_ref[...][None, :] - qtot) / wtot
        row = lax.broadcasted_iota(jnp.int32, (_CP, nsys), 0)
        f_ref[...] = jnp.where(row < _NC, f, 0.0)


def _sc_body(wi_hbm, qt_hbm, bi_hbm, f_hbm, out_hbm,
             bi_v, wi_v, qt_v, f_v, s0, s1, s2, s3):
    apw = wi_v.shape[1]
    wid = lax.axis_index("s") * 2 + lax.axis_index("c")
    base = wid * apw
    c0 = pltpu.async_copy(f_hbm, f_v, s0)
    c1 = pltpu.async_copy(bi_hbm.at[pl.ds(base, apw)], bi_v, s1)
    c2 = pltpu.async_copy(wi_hbm.at[:, pl.ds(base, apw)], wi_v, s2)
    c3 = pltpu.async_copy(qt_hbm.at[:, pl.ds(base, apw)], qt_v, s3)
    c0.wait()
    c1.wait()
    c2.wait()
    c3.wait()

    cvecs = [jnp.full((_L,), c, jnp.int32) for c in range(_NC)]

    @plsc.parallel_loop(0, apw, step=_L, unroll=4)
    def _combine(i):
        sl = pl.ds(i, _L)
        bi_vec = bi_v[sl]
        for c in range(_NC):
            fvals = plsc.load_gather(f_v, [cvecs[c], bi_vec])
            qt_v[c, sl] = qt_v[c, sl] + wi_v[c, sl] * fvals

    pltpu.sync_copy(qt_v, out_hbm.at[:, pl.ds(base, apw)])


def kernel(embedding, coordinates, batch_index, natoms, total_charge,
           W_wi, b_wi, W_qi, b_qi):
    n, d = embedding.shape
    nsys = natoms.shape[0]
    nc = W_wi.shape[1]
    apw = n // _NW

    wt = jnp.zeros((2 * _CP, d), jnp.float32)
    wt = wt.at[:nc, :].set(W_wi.T).at[_CP:_CP + nc, :].set(W_qi.T)
    b_col = jnp.zeros((2 * _CP, 1), jnp.float32)
    b_col = b_col.at[:nc, 0].set(b_wi).at[_CP:_CP + nc, 0].set(b_qi)

    nb = n // _BN
    wi_t, qt_t, f_t = pl.pallas_call(
        _tc_body,
        grid=(nb,),
        in_specs=[
            pl.BlockSpec((_BN, d), lambda b: (b, 0)),
            pl.BlockSpec((2 * _CP, d), lambda b: (0, 0)),
            pl.BlockSpec((2 * _CP, 1), lambda b: (0, 0)),
            pl.BlockSpec((_BN,), lambda b: (b,)),
            pl.BlockSpec((nsys,), lambda b: (0,)),
        ],
        out_specs=[
            pl.BlockSpec((_NC, _BN), lambda b: (0, b)),
            pl.BlockSpec((_NC, _BN), lambda b: (0, b)),
            pl.BlockSpec((_CP, nsys), lambda b: (0, 0)),
        ],
        out_shape=[
            jax.ShapeDtypeStruct((_NC, n), jnp.float32),
            jax.ShapeDtypeStruct((_NC, n), jnp.float32),
            jax.ShapeDtypeStruct((_CP, nsys), jnp.float32),
        ],
        scratch_shapes=[pltpu.VMEM((2 * _CP, nsys), jnp.float32)],
    )(embedding, wt, b_col, batch_index, total_charge)

    sc_fn = pl.kernel(
        _sc_body,
        out_type=jax.ShapeDtypeStruct((_NC, n), jnp.float32),
        mesh=plsc.VectorSubcoreMesh(core_axis_name="c", subcore_axis_name="s"),
        compiler_params=pltpu.CompilerParams(needs_layout_passes=False),
        scratch_types=[
            pltpu.VMEM((apw,), jnp.int32),
            pltpu.VMEM((_NC, apw), jnp.float32),
            pltpu.VMEM((_NC, apw), jnp.float32),
            pltpu.VMEM((_CP, nsys), jnp.float32),
            pltpu.SemaphoreType.DMA,
            pltpu.SemaphoreType.DMA,
            pltpu.SemaphoreType.DMA,
            pltpu.SemaphoreType.DMA,
        ],
    )
    q_t = sc_fn(wi_t, qt_t, batch_index, f_t)
    return q_t.T


# speedup vs baseline: 1.0390x; 1.0241x over previous
import jax
import jax.numpy as jnp
from jax import lax
from jax.experimental import pallas as pl
from jax.experimental.pallas import tpu as pltpu
from jax.experimental.pallas import tpu_sc as plsc

_BN = 8192
_CP = 16
_NC = 10
_NW = 32
_L = 16


def _tc_body(emb_ref, wt_ref, b_ref, bi_ref, qtot_ref,
             wi_ref, qt_ref, f_ref, acc_ref):
    b = pl.program_id(0)
    nb = pl.num_programs(0)
    nsys = qtot_ref.shape[0]
    bn = emb_ref.shape[0]
    y = lax.dot_general(wt_ref[...], emb_ref[...], (((1,), (1,)), ((), ())),
                        preferred_element_type=jnp.float32)
    y = y + b_ref[...]
    yw = y[:_CP, :]
    wi = jnp.maximum(yw, 0.0) + jnp.log1p(jnp.exp(-jnp.abs(yw)))
    qt = y[_CP:, :]
    wi_ref[...] = wi[:_NC, :]
    qt_ref[...] = qt[:_NC, :]

    sysid = lax.broadcasted_iota(jnp.int32, (nsys, bn), 0)
    masks_t = (bi_ref[...][None, :] == sysid).astype(jnp.float32)
    ywq = jnp.concatenate([wi, qt], axis=0)
    part = lax.dot_general(ywq, masks_t, (((1,), (1,)), ((), ())),
                           preferred_element_type=jnp.float32)

    @pl.when(b == 0)
    def _init():
        acc_ref[...] = part

    @pl.when(b != 0)
    def _accum():
        acc_ref[...] += part

    @pl.when(b == nb - 1)
    def _finish():
        acc = acc_ref[...]
        wtot = acc[:_CP, :]
        qtot = acc[_CP:, :]
        f = (qtot_ref[...][None, :] - qtot) / wtot
        row = lax.broadcasted_iota(jnp.int32, (_CP, nsys), 0)
        f_ref[...] = jnp.where(row < _NC, f, 0.0)


def _sc_body(wi_hbm, qt_hbm, bi_hbm, f_hbm, out_hbm,
             bi_v, wi_v, qt_v, f_v, s0, s1, s2, s3):
    apw = wi_v.shape[1]
    wid = lax.axis_index("s") * 2 + lax.axis_index("c")
    base = wid * apw
    c0 = pltpu.async_copy(f_hbm, f_v, s0)
    c1 = pltpu.async_copy(bi_hbm.at[pl.ds(base, apw)], bi_v, s1)
    c2 = pltpu.async_copy(wi_hbm.at[:, pl.ds(base, apw)], wi_v, s2)
    c3 = pltpu.async_copy(qt_hbm.at[:, pl.ds(base, apw)], qt_v, s3)
    c0.wait()
    c1.wait()
    c2.wait()
    c3.wait()

    cvecs = [jnp.full((_L,), c, jnp.int32) for c in range(_NC)]

    @plsc.parallel_loop(0, apw, step=_L, unroll=4)
    def _combine(i):
        sl = pl.ds(i, _L)
        bi_vec = bi_v[sl]
        for c in range(_NC):
            fvals = plsc.load_gather(f_v, [cvecs[c], bi_vec])
            qt_v[c, sl] = qt_v[c, sl] + wi_v[c, sl] * fvals

    pltpu.sync_copy(qt_v, out_hbm.at[:, pl.ds(base, apw)])


def kernel(embedding, coordinates, batch_index, natoms, total_charge,
           W_wi, b_wi, W_qi, b_qi):
    n, d = embedding.shape
    nsys = natoms.shape[0]
    nc = W_wi.shape[1]
    apw = n // _NW

    wt = jnp.zeros((2 * _CP, d), jnp.float32)
    wt = wt.at[:nc, :].set(W_wi.T).at[_CP:_CP + nc, :].set(W_qi.T)
    b_col = jnp.zeros((2 * _CP, 1), jnp.float32)
    b_col = b_col.at[:nc, 0].set(b_wi).at[_CP:_CP + nc, 0].set(b_qi)

    nb = n // _BN
    wi_t, qt_t, f_t = pl.pallas_call(
        _tc_body,
        grid=(nb,),
        in_specs=[
            pl.BlockSpec((_BN, d), lambda b: (b, 0)),
            pl.BlockSpec((2 * _CP, d), lambda b: (0, 0)),
            pl.BlockSpec((2 * _CP, 1), lambda b: (0, 0)),
            pl.BlockSpec((_BN,), lambda b: (b,)),
            pl.BlockSpec((nsys,), lambda b: (0,)),
        ],
        out_specs=[
            pl.BlockSpec((_NC, _BN), lambda b: (0, b)),
            pl.BlockSpec((_NC, _BN), lambda b: (0, b)),
            pl.BlockSpec((_CP, nsys), lambda b: (0, 0)),
        ],
        out_shape=[
            jax.ShapeDtypeStruct((_NC, n), jnp.float32),
            jax.ShapeDtypeStruct((_NC, n), jnp.float32),
            jax.ShapeDtypeStruct((_CP, nsys), jnp.float32),
        ],
        scratch_shapes=[pltpu.VMEM((2 * _CP, nsys), jnp.float32)],
    )(embedding, wt, b_col, batch_index, total_charge)

    sc_fn = pl.kernel(
        _sc_body,
        out_type=jax.ShapeDtypeStruct((_NC, n), jnp.float32),
        mesh=plsc.VectorSubcoreMesh(core_axis_name="c", subcore_axis_name="s"),
        compiler_params=pltpu.CompilerParams(needs_layout_passes=False),
        scratch_types=[
            pltpu.VMEM((apw,), jnp.int32),
            pltpu.VMEM((_NC, apw), jnp.float32),
            pltpu.VMEM((_NC, apw), jnp.float32),
            pltpu.VMEM((_CP, nsys), jnp.float32),
            pltpu.SemaphoreType.DMA,
            pltpu.SemaphoreType.DMA,
            pltpu.SemaphoreType.DMA,
            pltpu.SemaphoreType.DMA,
        ],
    )
    q_t = sc_fn(wi_t, qt_t, batch_index, f_t)
    return q_t.T


# speedup vs baseline: 1.0665x; 1.0265x over previous
import jax
import jax.numpy as jnp
from jax import lax
from jax.experimental import pallas as pl
from jax.experimental.pallas import tpu as pltpu
from jax.experimental.pallas import tpu_sc as plsc

_BN = 8192
_CP = 16
_NC = 10
_NW = 32
_L = 16


def _tc_body(emb_ref, wt_ref, b_ref, bi_ref, qtot_ref,
             wi_ref, qt_ref, f_ref, acc_ref):
    b = pl.program_id(0)
    nb = pl.num_programs(0)
    nsys = qtot_ref.shape[0]
    bn = emb_ref.shape[0]
    y = lax.dot_general(wt_ref[...], emb_ref[...], (((1,), (1,)), ((), ())),
                        preferred_element_type=jnp.float32)
    y = y + b_ref[...]
    yw = y[:_CP, :]
    wi = jnp.maximum(yw, 0.0) + jnp.log1p(jnp.exp(-jnp.abs(yw)))
    qt = y[_CP:, :]
    wi_ref[...] = wi[:_NC, :]
    qt_ref[...] = qt[:_NC, :]

    sysid = lax.broadcasted_iota(jnp.int32, (nsys, bn), 0)
    masks_t = (bi_ref[...][None, :] == sysid).astype(jnp.float32)
    ywq = jnp.concatenate([wi, qt], axis=0)
    part = lax.dot_general(ywq, masks_t, (((1,), (1,)), ((), ())),
                           preferred_element_type=jnp.float32)

    @pl.when(b == 0)
    def _init():
        acc_ref[...] = part

    @pl.when(b != 0)
    def _accum():
        acc_ref[...] += part

    @pl.when(b == nb - 1)
    def _finish():
        acc = acc_ref[...]
        wtot = acc[:_CP, :]
        qtot = acc[_CP:, :]
        f = (qtot_ref[...][None, :] - qtot) / wtot
        row = lax.broadcasted_iota(jnp.int32, (_CP, nsys), 0)
        f_ref[...] = jnp.where(row < _NC, f, 0.0)


def _sc_body(wi_hbm, qt_hbm, bi_hbm, f_hbm, out_hbm,
             bi_v, wi_v, qt_v, f_v, s0, s1, s2, s3):
    apw = wi_v.shape[1]
    wid = lax.axis_index("s") * 2 + lax.axis_index("c")
    base = wid * apw
    c0 = pltpu.async_copy(f_hbm, f_v, s0)
    c1 = pltpu.async_copy(bi_hbm.at[pl.ds(base, apw)], bi_v, s1)
    c2 = pltpu.async_copy(wi_hbm.at[:, pl.ds(base, apw)], wi_v, s2)
    c3 = pltpu.async_copy(qt_hbm.at[:, pl.ds(base, apw)], qt_v, s3)
    c0.wait()
    c1.wait()
    c2.wait()
    c3.wait()

    cvecs = [jnp.full((_L,), c, jnp.int32) for c in range(_NC)]

    @plsc.parallel_loop(0, apw, step=_L, unroll=4)
    def _combine(i):
        sl = pl.ds(i, _L)
        bi_vec = bi_v[sl]
        for c in range(_NC):
            fvals = plsc.load_gather(f_v, [cvecs[c], bi_vec])
            qt_v[c, sl] = qt_v[c, sl] + wi_v[c, sl] * fvals

    pltpu.sync_copy(qt_v, out_hbm.at[:, pl.ds(base, apw)])


def kernel(embedding, coordinates, batch_index, natoms, total_charge,
           W_wi, b_wi, W_qi, b_qi):
    n, d = embedding.shape
    nsys = natoms.shape[0]
    nc = W_wi.shape[1]
    apw = n // _NW

    zw = jnp.zeros((d, _CP - nc), jnp.float32)
    wt = jnp.concatenate([W_wi, zw, W_qi, zw], axis=1).T
    zb = jnp.zeros((_CP - nc,), jnp.float32)
    b_col = jnp.concatenate([b_wi, zb, b_qi, zb])[:, None]

    nb = n // _BN
    wi_t, qt_t, f_t = pl.pallas_call(
        _tc_body,
        grid=(nb,),
        in_specs=[
            pl.BlockSpec((_BN, d), lambda b: (b, 0)),
            pl.BlockSpec((2 * _CP, d), lambda b: (0, 0)),
            pl.BlockSpec((2 * _CP, 1), lambda b: (0, 0)),
            pl.BlockSpec((_BN,), lambda b: (b,)),
            pl.BlockSpec((nsys,), lambda b: (0,)),
        ],
        out_specs=[
            pl.BlockSpec((_NC, _BN), lambda b: (0, b)),
            pl.BlockSpec((_NC, _BN), lambda b: (0, b)),
            pl.BlockSpec((_CP, nsys), lambda b: (0, 0)),
        ],
        out_shape=[
            jax.ShapeDtypeStruct((_NC, n), jnp.float32),
            jax.ShapeDtypeStruct((_NC, n), jnp.float32),
            jax.ShapeDtypeStruct((_CP, nsys), jnp.float32),
        ],
        scratch_shapes=[pltpu.VMEM((2 * _CP, nsys), jnp.float32)],
    )(embedding, wt, b_col, batch_index, total_charge)

    sc_fn = pl.kernel(
        _sc_body,
        out_type=jax.ShapeDtypeStruct((_NC, n), jnp.float32),
        mesh=plsc.VectorSubcoreMesh(core_axis_name="c", subcore_axis_name="s"),
        compiler_params=pltpu.CompilerParams(needs_layout_passes=False),
        scratch_types=[
            pltpu.VMEM((apw,), jnp.int32),
            pltpu.VMEM((_NC, apw), jnp.float32),
            pltpu.VMEM((_NC, apw), jnp.float32),
            pltpu.VMEM((_CP, nsys), jnp.float32),
            pltpu.SemaphoreType.DMA,
            pltpu.SemaphoreType.DMA,
            pltpu.SemaphoreType.DMA,
            pltpu.SemaphoreType.DMA,
        ],
    )
    q_t = sc_fn(wi_t, qt_t, batch_index, f_t)
    return q_t.T


# speedup vs baseline: 1.0667x; 1.0002x over previous
import jax
import jax.numpy as jnp
from jax import lax
from jax.experimental import pallas as pl
from jax.experimental.pallas import tpu as pltpu
from jax.experimental.pallas import tpu_sc as plsc

_BN = 8192
_CP = 16
_NC = 10
_NW = 32
_L = 16


def _tc_body(emb_ref, wt_ref, b_ref, bi_ref, qtot_ref,
             wi_ref, qt_ref, f_ref, acc_ref):
    b = pl.program_id(0)
    nb = pl.num_programs(0)
    nsys = qtot_ref.shape[0]
    bn = emb_ref.shape[0]
    y = lax.dot_general(wt_ref[...], emb_ref[...], (((1,), (1,)), ((), ())),
                        preferred_element_type=jnp.float32)
    y = y + b_ref[...]
    yw = y[:_CP, :]
    wi = jnp.maximum(yw, 0.0) + jnp.log1p(jnp.exp(-jnp.abs(yw)))
    qt = y[_CP:, :]
    wi_ref[...] = wi[:_NC, :]
    qt_ref[...] = qt[:_NC, :]

    sysid = lax.broadcasted_iota(jnp.int32, (nsys, bn), 0)
    masks_t = (bi_ref[...][None, :] == sysid).astype(jnp.float32)
    ywq = jnp.concatenate([wi, qt], axis=0)
    part = lax.dot_general(ywq, masks_t, (((1,), (1,)), ((), ())),
                           preferred_element_type=jnp.float32)

    @pl.when(b == 0)
    def _init():
        acc_ref[...] = part

    @pl.when(b != 0)
    def _accum():
        acc_ref[...] += part

    @pl.when(b == nb - 1)
    def _finish():
        acc = acc_ref[...]
        wtot = acc[:_CP, :]
        qtot = acc[_CP:, :]
        f = (qtot_ref[...][None, :] - qtot) / wtot
        row = lax.broadcasted_iota(jnp.int32, (_CP, nsys), 0)
        f_ref[...] = jnp.where(row < _NC, f, 0.0)


def _sc_body(wi_hbm, qt_hbm, bi_hbm, f_hbm, out_hbm,
             bi_v, wi_v, qt_v, f_v, qo_v, s0, s1, s2, s3):
    apw = wi_v.shape[1]
    wid = lax.axis_index("s") * 2 + lax.axis_index("c")
    base = wid * apw
    c0 = pltpu.async_copy(f_hbm, f_v, s0)
    c1 = pltpu.async_copy(bi_hbm.at[pl.ds(base, apw)], bi_v, s1)
    c2 = pltpu.async_copy(wi_hbm.at[:, pl.ds(base, apw)], wi_v, s2)
    c3 = pltpu.async_copy(qt_hbm.at[:, pl.ds(base, apw)], qt_v, s3)
    c0.wait()
    c1.wait()
    c2.wait()
    c3.wait()

    cvecs = [jnp.full((_L,), c, jnp.int32) for c in range(_NC)]

    @plsc.parallel_loop(0, apw, step=_L, unroll=4)
    def _combine(i):
        sl = pl.ds(i, _L)
        bi_vec = bi_v[sl]
        for c in range(_NC):
            fvals = plsc.load_gather(f_v, [cvecs[c], bi_vec])
            qo_v[c, sl] = qt_v[c, sl] + wi_v[c, sl] * fvals

    pltpu.sync_copy(qo_v, out_hbm.at[:, pl.ds(base, apw)])


def kernel(embedding, coordinates, batch_index, natoms, total_charge,
           W_wi, b_wi, W_qi, b_qi):
    n, d = embedding.shape
    nsys = natoms.shape[0]
    nc = W_wi.shape[1]
    apw = n // _NW

    zw = jnp.zeros((d, _CP - nc), jnp.float32)
    wt = jnp.concatenate([W_wi, zw, W_qi, zw], axis=1).T
    zb = jnp.zeros((_CP - nc,), jnp.float32)
    b_col = jnp.concatenate([b_wi, zb, b_qi, zb])[:, None]

    nb = n // _BN
    wi_t, qt_t, f_t = pl.pallas_call(
        _tc_body,
        grid=(nb,),
        in_specs=[
            pl.BlockSpec((_BN, d), lambda b: (b, 0)),
            pl.BlockSpec((2 * _CP, d), lambda b: (0, 0)),
            pl.BlockSpec((2 * _CP, 1), lambda b: (0, 0)),
            pl.BlockSpec((_BN,), lambda b: (b,)),
            pl.BlockSpec((nsys,), lambda b: (0,)),
        ],
        out_specs=[
            pl.BlockSpec((_NC, _BN), lambda b: (0, b)),
            pl.BlockSpec((_NC, _BN), lambda b: (0, b)),
            pl.BlockSpec((_CP, nsys), lambda b: (0, 0)),
        ],
        out_shape=[
            jax.ShapeDtypeStruct((_NC, n), jnp.float32),
            jax.ShapeDtypeStruct((_NC, n), jnp.float32),
            jax.ShapeDtypeStruct((_CP, nsys), jnp.float32),
        ],
        scratch_shapes=[pltpu.VMEM((2 * _CP, nsys), jnp.float32)],
    )(embedding, wt, b_col, batch_index, total_charge)

    sc_fn = pl.kernel(
        _sc_body,
        out_type=jax.ShapeDtypeStruct((_NC, n), jnp.float32),
        mesh=plsc.VectorSubcoreMesh(core_axis_name="c", subcore_axis_name="s"),
        compiler_params=pltpu.CompilerParams(needs_layout_passes=False),
        scratch_types=[
            pltpu.VMEM((apw,), jnp.int32),
            pltpu.VMEM((_NC, apw), jnp.float32),
            pltpu.VMEM((_NC, apw), jnp.float32),
            pltpu.VMEM((_CP, nsys), jnp.float32),
            pltpu.VMEM((_NC, apw), jnp.float32),
            pltpu.SemaphoreType.DMA,
            pltpu.SemaphoreType.DMA,
            pltpu.SemaphoreType.DMA,
            pltpu.SemaphoreType.DMA,
        ],
    )
    q_t = sc_fn(wi_t, qt_t, batch_index, f_t)
    return q_t.T
